# Initial kernel scaffold; baseline (speedup 1.0000x reference)
import jax
import jax.numpy as jnp
from jax.experimental import pallas as pl


def kernel(x, label_length, table, ctr_W1, ctr_b1, ctr_W2, ctr_b2, ctr_Wo, ctr_bo, cvr_W1, cvr_b1, cvr_W2, cvr_b2, cvr_Wo, cvr_bo):
    raise NotImplementedError("write your pallas kernel here")


# speedup vs baseline: 4.3778x; 4.3778x over previous
import functools

import jax
import jax.numpy as jnp
from jax import lax
from jax.experimental import pallas as pl
from jax.experimental.pallas import tpu as pltpu
from jax.experimental.pallas import tpu_sc as plsc


@functools.lru_cache(maxsize=None)
def _make_gather(T, F, D):
    NW = 32
    TPW = T // NW
    C = 4
    IDS = F * C
    NCH = TPW // C

    mesh = plsc.VectorSubcoreMesh(core_axis_name="c", subcore_axis_name="s")

    @functools.partial(
        pl.kernel,
        out_type=jax.ShapeDtypeStruct((T, D), jnp.float32),
        mesh=mesh,
        scratch_types=[
            pltpu.VMEM((TPW * F,), jnp.int32),
            pltpu.VMEM((TPW, D), jnp.float32),
            pltpu.VMEM((IDS, D), jnp.float32),
            pltpu.VMEM((IDS, D), jnp.float32),
            pltpu.SemaphoreType.DMA,
            pltpu.SemaphoreType.DMA,
        ],
        compiler_params=pltpu.CompilerParams(use_tc_tiling_on_sc=False),
    )
    def gather_kernel(x_hbm, table_hbm, out_hbm, idx_v, emb_v, buf0, buf1,
                      sem0, sem1):
        wid = lax.axis_index("s") * 2 + lax.axis_index("c")
        base_t = wid * TPW

        pltpu.sync_copy(x_hbm.at[pl.ds(base_t * F, TPW * F)], idx_v)

        bufs = (buf0, buf1)
        sems = (sem0, sem1)

        def start(ch, b):
            pltpu.make_async_copy(
                table_hbm.at[idx_v.at[pl.ds(ch * IDS, IDS)]],
                bufs[b], sems[b]).start()

        def wait(b):
            pltpu.make_async_copy(
                table_hbm.at[idx_v.at[pl.ds(0, IDS)]],
                bufs[b], sems[b]).wait()

        def accumulate(ch, b):
            buf = bufs[b]
            for t in range(C):
                acc = buf[t * F, :]
                for f in range(1, F):
                    acc = acc + buf[t * F + f, :]
                emb_v[ch * C + t, :] = acc

        start(0, 0)
        start(1, 1)

        @pl.loop(0, NCH // 2)
        def _(g):
            ch = g * 2
            wait(0)
            accumulate(ch, 0)

            @pl.when(ch + 2 < NCH)
            def _():
                start(ch + 2, 0)

            wait(1)
            accumulate(ch + 1, 1)

            @pl.when(ch + 3 < NCH)
            def _():
                start(ch + 3, 1)

        pltpu.sync_copy(emb_v, out_hbm.at[pl.ds(base_t, TPW)])

    return gather_kernel


def _mlp_body(emb_ref, w1_ref, b1_ref, w2_ref, b2_ref, wo_ref, bo_ref,
              out_ref):
    h = emb_ref[...]
    h1 = jnp.maximum(
        jnp.dot(h, w1_ref[...], preferred_element_type=jnp.float32)
        + b1_ref[...], 0.0)
    h2 = jnp.maximum(
        jnp.dot(h1, w2_ref[...], preferred_element_type=jnp.float32)
        + b2_ref[...], 0.0)
    o = jnp.dot(h2, wo_ref[...], preferred_element_type=jnp.float32) \
        + bo_ref[...]
    p = jax.nn.sigmoid(o)
    pctr = p[:, 0:1]
    out_ref[...] = jnp.concatenate([pctr, pctr * p[:, 1:2]], axis=1)


@functools.lru_cache(maxsize=None)
def _make_mlp(T, D, H1c, H2c):
    BT = 2048
    grid = (T // BT,)

    def full(shape):
        return pl.BlockSpec(shape, lambda i: (0, 0))

    return pl.pallas_call(
        _mlp_body,
        grid=grid,
        in_specs=[
            pl.BlockSpec((BT, D), lambda i: (i, 0)),
            full((D, H1c)),
            full((1, H1c)),
            full((H1c, H2c)),
            full((1, H2c)),
            full((H2c, 2)),
            full((1, 2)),
        ],
        out_specs=pl.BlockSpec((BT, 2), lambda i: (i, 0)),
        out_shape=jax.ShapeDtypeStruct((T, 2), jnp.float32),
    )


def kernel(x, label_length, table,
           ctr_W1, ctr_b1, ctr_W2, ctr_b2, ctr_Wo, ctr_bo,
           cvr_W1, cvr_b1, cvr_W2, cvr_b2, cvr_Wo, cvr_bo):
    T, F = x.shape
    D = table.shape[1]
    H1 = ctr_W1.shape[1]
    H2 = ctr_W2.shape[1]

    emb = _make_gather(T, F, D)(x.reshape(T * F), table)

    z12 = jnp.zeros((H1, H2), jnp.float32)
    z2o = jnp.zeros((H2, 1), jnp.float32)
    W1 = jnp.concatenate([ctr_W1, cvr_W1], axis=1)
    b1 = jnp.concatenate([ctr_b1, cvr_b1])[None, :]
    W2 = jnp.concatenate(
        [jnp.concatenate([ctr_W2, z12], axis=1),
         jnp.concatenate([z12, cvr_W2], axis=1)], axis=0)
    b2 = jnp.concatenate([ctr_b2, cvr_b2])[None, :]
    Wo = jnp.concatenate(
        [jnp.concatenate([ctr_Wo, z2o], axis=1),
         jnp.concatenate([z2o, cvr_Wo], axis=1)], axis=0)
    bo = jnp.concatenate([ctr_bo, cvr_bo])[None, :]

    return _make_mlp(T, D, 2 * H1, 2 * H2)(emb, W1, b1, W2, b2, Wo, bo)
